# baseline (speedup 1.0000x reference)
import dataclasses
import functools

import jax
import jax.numpy as jnp
from jax import lax
from jax.experimental import pallas as pl
from jax.experimental.pallas import tpu as pltpu
from jax.experimental.pallas import tpu_sc as plsc

_N = 10000
_D = 128
_K = 5000
_NW = 32
_PER = 160
_CTAIL = _K - 8
_G = 4
_XG = 8


def _compiler_params():
  cp = pltpu.CompilerParams()
  if "needs_layout_passes" in pltpu.CompilerParams.__dataclass_fields__:
    cp = dataclasses.replace(cp, needs_layout_passes=False)
  return cp


def _sc_gather(A, X, idx, values):
  mesh = plsc.VectorSubcoreMesh(core_axis_name="c", subcore_axis_name="s")

  @functools.partial(
      pl.kernel,
      compiler_params=_compiler_params(),
      out_type=(jax.ShapeDtypeStruct((_K, _K), jnp.float32),
                jax.ShapeDtypeStruct((_K, _D), jnp.float32)),
      mesh=mesh,
      scratch_types=(
          [pltpu.VMEM((_K,), jnp.int32)]
          + [pltpu.VMEM((_N,), jnp.float32)] * 8
          + [pltpu.VMEM((_K,), jnp.float32)] * 8
          + [
              pltpu.VMEM((2 * _XG * _D,), jnp.float32),
              pltpu.VMEM((_PER,), jnp.float32),
              pltpu.SemaphoreType.DMA,
              pltpu.SemaphoreType.DMA,
              pltpu.SemaphoreType.DMA,
              pltpu.SemaphoreType.DMA,
              pltpu.SemaphoreType.DMA,
              pltpu.SemaphoreType.DMA,
          ]
      ),
  )
  def k(A_hbm, X_hbm, idx_hbm, val_hbm, outA_hbm, outX_hbm,
        idx_v, a00, a01, a02, a03, a10, a11, a12, a13,
        o00, o01, o02, o03, o10, o11, o12, o13, xbuf, val_v,
        semA0, semA1, semW0, semW1, semX, semXW):
    arow = ((a00, a01, a02, a03), (a10, a11, a12, a13))
    orow = ((o00, o01, o02, o03), (o10, o11, o12, o13))
    semA = (semA0, semA1)
    semW = (semW0, semW1)
    wid = lax.axis_index("s") * 2 + lax.axis_index("c")
    base = wid * _PER
    pltpu.sync_copy(idx_hbm, idx_v)
    pltpu.sync_copy(val_hbm.at[pl.ds(base, _PER)], val_v)
    iota = lax.iota(jnp.int32, 16)
    tail_mask = iota < 8

    def _jclamp(j):
      return jnp.minimum(j, _K - 1 - base)

    def _src(j):
      v = plsc.load_gather(idx_v, [jnp.full((16,), base + _jclamp(j),
                                            jnp.int32)])
      return jnp.max(v)

    def _fetch(j, buf):
      for u in range(_G):
        pltpu.make_async_copy(A_hbm.at[_src(j + u)], arow[buf][u],
                              semA[buf]).start()

    def _process(j, buf):
      @pl.when(j >= 2 * _G)
      def _():
        _drain(buf)

      for u in range(_G):
        pltpu.make_async_copy(A_hbm.at[0], arow[buf][u], semA[buf]).wait()

      @plsc.parallel_loop(0, _CTAIL, step=16, unroll=8)
      def _cols(c):
        cols = idx_v[pl.ds(c, 16)]
        for u in range(_G):
          orow[buf][u][pl.ds(c, 16)] = plsc.load_gather(arow[buf][u], [cols])

      cols_raw = plsc.load_gather(idx_v, [jnp.minimum(iota + _CTAIL, _K - 1)])
      cols = jnp.where(tail_mask, cols_raw, 0)
      pos = iota + _CTAIL
      for u in range(_G):
        plsc.store_scatter(orow[buf][u], [pos],
                           plsc.load_gather(arow[buf][u], [cols]),
                           mask=tail_mask)
      for u in range(_G):
        pltpu.make_async_copy(orow[buf][u],
                              outA_hbm.at[base + _jclamp(j + u)],
                              semW[buf]).start()

    def _drain(buf):
      for u in range(_G):
        pltpu.make_async_copy(orow[buf][u], outA_hbm.at[0], semW[buf]).wait()

    _fetch(0, 0)
    _fetch(_G, 1)

    @pl.loop(0, _PER, step=_XG)
    def _xgroup(g):
      slot = lax.rem(lax.div(g, _XG), 2) * (_XG * _D)

      @plsc.parallel_loop(0, _XG)
      def _xin(u):
        pltpu.make_async_copy(X_hbm.at[_src(g + u)],
                              xbuf.at[pl.ds(slot + u * _D, _D)], semX).start()

      @pl.loop(0, _XG)
      def _xscale(u):
        pltpu.make_async_copy(X_hbm.at[0], xbuf.at[pl.ds(slot + u * _D, _D)],
                              semX).wait()
        vv = plsc.load_gather(val_v, [jnp.full((16,), _jclamp(g + u),
                                               jnp.int32)])

        @plsc.parallel_loop(0, _D, step=16, unroll=8)
        def _xs(c):
          xbuf[pl.ds(slot + u * _D + c, 16)] = (
              xbuf[pl.ds(slot + u * _D + c, 16)] * vv)

      @pl.when(g > 0)
      def _():
        @pl.loop(0, _XG)
        def _xdrain(u):
          pltpu.make_async_copy(xbuf.at[pl.ds(u * _D, _D)],
                                outX_hbm.at[0], semXW).wait()

      @pl.loop(0, _XG)
      def _xout(u):
        pltpu.make_async_copy(xbuf.at[pl.ds(slot + u * _D, _D)],
                              outX_hbm.at[base + _jclamp(g + u)],
                              semXW).start()

    @pl.loop(0, _XG)
    def _xdrain_last(u):
      pltpu.make_async_copy(xbuf.at[pl.ds(u * _D, _D)], outX_hbm.at[0],
                            semXW).wait()

    @pl.loop(0, _PER, step=2 * _G)
    def _oct(t):
      _process(t, 0)

      @pl.when(t + 2 * _G < _PER)
      def _():
        _fetch(t + 2 * _G, 0)

      _process(t + _G, 1)

      @pl.when(t + 3 * _G < _PER)
      def _():
        _fetch(t + 3 * _G, 1)

    _drain(0)
    _drain(1)

  return k(A, X, idx, values)


def kernel(A, X, W, b):
  scores = X @ W + b
  scores = jnp.squeeze(scores)
  scores = jax.nn.sigmoid(scores / 100.0)
  values, idx = jax.lax.top_k(scores, _K)
  new_A, new_X = _sc_gather(A, X, idx, values)
  return (new_A, new_X, idx)


# speedup vs baseline: 1.0515x; 1.0515x over previous
import dataclasses
import functools

import jax
import jax.numpy as jnp
from jax import lax
from jax.experimental import pallas as pl
from jax.experimental.pallas import tpu as pltpu
from jax.experimental.pallas import tpu_sc as plsc

_N = 10000
_D = 128
_K = 5000
_NW = 32
_PER = 160
_IPAD = 5120
_CTAIL = _K - 8
_XG = 8


def _compiler_params():
  cp = pltpu.CompilerParams()
  if "needs_layout_passes" in pltpu.CompilerParams.__dataclass_fields__:
    cp = dataclasses.replace(cp, needs_layout_passes=False)
  return cp


def _sc_gather(A, X, idx_pad, val_pad):
  mesh = plsc.VectorSubcoreMesh(core_axis_name="c", subcore_axis_name="s")

  @functools.partial(
      pl.kernel,
      compiler_params=_compiler_params(),
      out_type=(jax.ShapeDtypeStruct((_K, _K), jnp.float32),
                jax.ShapeDtypeStruct((_K, _D), jnp.float32)),
      mesh=mesh,
      scratch_types=[
          pltpu.VMEM((_K,), jnp.int32),
          pltpu.VMEM((_N,), jnp.float32),
          pltpu.VMEM((_N,), jnp.float32),
          pltpu.VMEM((_N,), jnp.float32),
          pltpu.VMEM((_N,), jnp.float32),
          pltpu.VMEM((_N,), jnp.float32),
          pltpu.VMEM((_N,), jnp.float32),
          pltpu.VMEM((_K,), jnp.float32),
          pltpu.VMEM((_K,), jnp.float32),
          pltpu.VMEM((_K,), jnp.float32),
          pltpu.VMEM((_K,), jnp.float32),
          pltpu.VMEM((_K,), jnp.float32),
          pltpu.VMEM((_K,), jnp.float32),
          pltpu.VMEM((_PER * _D,), jnp.float32),
          pltpu.VMEM((_K,), jnp.float32),
          pltpu.SemaphoreType.DMA,
          pltpu.SemaphoreType.DMA,
          pltpu.SemaphoreType.DMA,
          pltpu.SemaphoreType.DMA,
          pltpu.SemaphoreType.DMA,
          pltpu.SemaphoreType.DMA,
          pltpu.SemaphoreType.DMA,
          pltpu.SemaphoreType.DMA,
      ],
  )
  def k(A_hbm, X_hbm, idx_hbm, val_hbm, outA_hbm, outX_hbm,
        idx_v, a00, a01, a10, a11, a20, a21, o00, o01, o10, o11, o20, o21,
        xbuf, val_v, semA0, semA1, semA2, semW0, semW1, semW2, semX, semXW):
    arow = ((a00, a01), (a10, a11), (a20, a21))
    orow = ((o00, o01), (o10, o11), (o20, o21))
    semA = (semA0, semA1, semA2)
    semW = (semW0, semW1, semW2)
    wid = lax.axis_index("s") * 2 + lax.axis_index("c")
    base = 156 * wid + 2 * jnp.minimum(wid, 4)
    nvalid = jnp.where(wid < 4, 158, 156)
    pltpu.sync_copy(idx_hbm, idx_v)
    pltpu.sync_copy(val_hbm, val_v)
    iota = lax.iota(jnp.int32, 16)
    tail_mask = iota < 8

    def _src_g(jg):
      v = plsc.load_gather(idx_v, [jnp.full((16,), jg, jnp.int32)])
      return jnp.max(v)

    def _src(j):
      return _src_g(base + j)

    def _fetch(j, buf):
      @pl.when(j < nvalid)
      def _():
        pltpu.make_async_copy(A_hbm.at[_src(j)], arow[buf][0],
                              semA[buf]).start()
        pltpu.make_async_copy(A_hbm.at[_src(j + 1)], arow[buf][1],
                              semA[buf]).start()

    def _process(j, buf):
      @pl.when(j < nvalid)
      def _():
        pltpu.make_async_copy(A_hbm.at[0], arow[buf][0], semA[buf]).wait()
        pltpu.make_async_copy(A_hbm.at[0], arow[buf][1], semA[buf]).wait()

        @plsc.parallel_loop(0, _CTAIL, step=16, unroll=8)
        def _cols(c):
          cols = idx_v[pl.ds(c, 16)]
          orow[buf][0][pl.ds(c, 16)] = plsc.load_gather(arow[buf][0], [cols])
          orow[buf][1][pl.ds(c, 16)] = plsc.load_gather(arow[buf][1], [cols])

        cols_raw = plsc.load_gather(
            idx_v, [jnp.minimum(iota + _CTAIL, _K - 1)])
        cols = jnp.where(tail_mask, cols_raw, 0)
        pos = iota + _CTAIL
        plsc.store_scatter(orow[buf][0], [pos],
                           plsc.load_gather(arow[buf][0], [cols]),
                           mask=tail_mask)
        plsc.store_scatter(orow[buf][1], [pos],
                           plsc.load_gather(arow[buf][1], [cols]),
                           mask=tail_mask)
        pltpu.make_async_copy(orow[buf][0], outA_hbm.at[base + j],
                              semW[buf]).start()
        pltpu.make_async_copy(orow[buf][1], outA_hbm.at[base + j + 1],
                              semW[buf]).start()

    def _drain(j, buf):
      @pl.when(j < nvalid)
      def _():
        pltpu.make_async_copy(orow[buf][0], outA_hbm.at[0], semW[buf]).wait()
        pltpu.make_async_copy(orow[buf][1], outA_hbm.at[0], semW[buf]).wait()

    _fetch(0, 0)
    _fetch(2, 1)
    _fetch(4, 2)

    @pl.loop(0, _PER, step=_XG)
    def _xgroup(g):
      @plsc.parallel_loop(0, _XG)
      def _xin(u):
        j = jnp.minimum(base + g + u, _K - 1)
        pltpu.make_async_copy(X_hbm.at[_src_g(j)],
                              xbuf.at[pl.ds((g + u) * _D, _D)], semX).start()

      @pl.loop(0, _XG)
      def _xscale(u):
        j = jnp.minimum(base + g + u, _K - 1)
        pltpu.make_async_copy(X_hbm.at[0], xbuf.at[pl.ds((g + u) * _D, _D)],
                              semX).wait()
        vv = plsc.load_gather(val_v, [jnp.full((16,), j, jnp.int32)])

        @plsc.parallel_loop(0, _D, step=16, unroll=8)
        def _xs(c):
          q = (g + u) * _D + c
          xbuf[pl.ds(q, 16)] = xbuf[pl.ds(q, 16)] * vv

      @pl.when(g > 0)
      def _():
        @pl.loop(0, _XG)
        def _xdrain(u):
          pltpu.make_async_copy(xbuf.at[pl.ds(u * _D, _D)],
                                outX_hbm.at[0], semXW).wait()

      @pl.loop(0, _XG)
      def _xout(u):
        j = jnp.minimum(base + g + u, _K - 1)
        pltpu.make_async_copy(xbuf.at[pl.ds((g + u) * _D, _D)],
                              outX_hbm.at[j], semXW).start()

    @pl.loop(0, _XG)
    def _xdrain_last(u):
      pltpu.make_async_copy(xbuf.at[pl.ds(u * _D, _D)], outX_hbm.at[0],
                            semXW).wait()

    @pl.loop(0, 162, step=6)
    def _hex(t):
      _process(t, 0)
      _fetch(t + 6, 0)
      _drain(t, 0)
      _process(t + 2, 1)
      _fetch(t + 8, 1)
      _drain(t + 2, 1)
      _process(t + 4, 2)
      _fetch(t + 10, 2)
      _drain(t + 4, 2)

  return k(A, X, idx_pad, val_pad)


def kernel(A, X, W, b):
  scores = X @ W + b
  scores = jnp.squeeze(scores)
  scores = jax.nn.sigmoid(scores / 100.0)
  values, idx = jax.lax.top_k(scores, _K)
  new_A, new_X = _sc_gather(A, X, idx, values)
  return (new_A, new_X, idx)
